# baseline (speedup 1.0000x reference)
import functools

import jax
import jax.numpy as jnp
from jax import lax
from jax.experimental import pallas as pl
from jax.experimental.pallas import tpu as pltpu
from jax.experimental.pallas import tpu_sc as plsc

V = 52
D = 64
NB = 16384
S = 50
NC, NS = 2, 16
NW = NC * NS
B_PER_W = NB // NW
GB = 4
L = 16

_mesh = plsc.VectorSubcoreMesh(core_axis_name="c", subcore_axis_name="s")


@functools.partial(
    pl.kernel,
    out_type=jax.ShapeDtypeStruct((S, D, NB), jnp.float32),
    mesh=_mesh,
    compiler_params=pltpu.CompilerParams(needs_layout_passes=False),
    scratch_types=[
        pltpu.VMEM((B_PER_W * S,), jnp.int32),
        pltpu.VMEM((V * D,), jnp.float32),
        pltpu.VMEM((D, 128), jnp.float32),
        pltpu.VMEM((D, 128), jnp.float32),
        pltpu.SemaphoreType.DMA,
        pltpu.SemaphoreType.DMA,
    ],
)
def _gather_kernel(idx_hbm, table_hbm, out_hbm, idx_v, tab_v,
                   stage0, stage1, ssem0, ssem1):
    wid = lax.axis_index("s") * NC + lax.axis_index("c")
    base = pl.multiple_of(wid * (B_PER_W * S), B_PER_W * S)
    pltpu.sync_copy(idx_hbm.at[pl.ds(base, B_PER_W * S)], idx_v)
    pltpu.sync_copy(table_hbm, tab_v)
    stage = (stage0, stage1)
    ssem = (ssem0, ssem1)

    iota = lax.iota(jnp.int32, L) * S

    def store(s, gb, half):
        b0 = (wid * GB + gb) * 128
        return pltpu.make_async_copy(
            stage[half], out_hbm.at[s, :, pl.ds(pl.multiple_of(b0, 128), 128)],
            ssem[half])

    def gb_body(gb, carry):
        def body(t, carry):
            for half in range(2):
                s = 2 * t + half

                @pl.when(t >= 1)
                def _(s=s, half=half):
                    store(s - 2, gb, half).wait()

                for g in range(8):
                    off0 = gb * (128 * S) + g * (L * S) + s
                    idx16 = plsc.load_gather(idx_v, [iota + off0])
                    rowb = idx16 * D
                    for d in range(D):
                        val = plsc.load_gather(tab_v, [rowb + d])
                        stage[half][d, pl.ds(g * L, L)] = val

                store(s, gb, half).start()
            return carry

        lax.fori_loop(0, S // 2, body, 0)
        store(S - 2, gb, 0).wait()
        store(S - 1, gb, 1).wait()
        return carry

    lax.fori_loop(0, GB, gb_body, 0)


def kernel(action_indices, embedding_table):
    flat_idx = action_indices.reshape(-1).astype(jnp.int32)
    out = _gather_kernel(flat_idx, embedding_table.reshape(-1))
    return jnp.transpose(out, (2, 0, 1))


# speedup vs baseline: 1.8444x; 1.8444x over previous
import functools

import jax
import jax.numpy as jnp
from jax import lax
from jax.experimental import pallas as pl
from jax.experimental.pallas import tpu as pltpu
from jax.experimental.pallas import tpu_sc as plsc

V = 52
D = 64
NB = 16384
S = 50
NC, NS = 2, 16
NW = NC * NS
B_PER_W = NB // NW
GB = 4
L = 16

_mesh = plsc.VectorSubcoreMesh(core_axis_name="c", subcore_axis_name="s")


@functools.partial(
    pl.kernel,
    out_type=jax.ShapeDtypeStruct((S, D, NB), jnp.float32),
    mesh=_mesh,
    compiler_params=pltpu.CompilerParams(needs_layout_passes=False),
    scratch_types=[
        pltpu.VMEM((B_PER_W * S,), jnp.int32),
        pltpu.VMEM((V * D,), jnp.float32),
        pltpu.VMEM((D, 128), jnp.float32),
        pltpu.VMEM((D, 128), jnp.float32),
        pltpu.SemaphoreType.DMA,
        pltpu.SemaphoreType.DMA,
    ],
)
def _gather_kernel(idx_hbm, table_hbm, out_hbm, idx_v, tab_v,
                   stage0, stage1, ssem0, ssem1):
    wid = lax.axis_index("s") * NC + lax.axis_index("c")
    base = pl.multiple_of(wid * (B_PER_W * S), B_PER_W * S)
    pltpu.sync_copy(idx_hbm.at[pl.ds(base, B_PER_W * S)], idx_v)
    pltpu.sync_copy(table_hbm, tab_v)
    stage = (stage0, stage1)
    ssem = (ssem0, ssem1)

    iota = lax.iota(jnp.int32, L) * S

    def store(s, gb, half):
        b0 = (wid * GB + gb) * 128
        return pltpu.make_async_copy(
            stage[half], out_hbm.at[s, :, pl.ds(pl.multiple_of(b0, 128), 128)],
            ssem[half])

    def gb_body(gb, carry):
        def body(t, carry):
            for half in range(2):
                s = 2 * t + half

                @pl.when(t >= 1)
                def _(s=s, half=half):
                    store(s - 2, gb, half).wait()

                for g in range(8):
                    off0 = gb * (128 * S) + g * (L * S) + s
                    idx16 = plsc.load_gather(idx_v, [iota + off0])
                    rowb = idx16 * D

                    @plsc.parallel_loop(0, D, unroll=8)
                    def _(d, g=g, half=half, rowb=rowb):
                        val = plsc.load_gather(tab_v, [rowb + d])
                        stage[half][d, pl.ds(g * L, L)] = val

                store(s, gb, half).start()
            return carry

        lax.fori_loop(0, S // 2, body, 0)
        store(S - 2, gb, 0).wait()
        store(S - 1, gb, 1).wait()
        return carry

    lax.fori_loop(0, GB, gb_body, 0)


def kernel(action_indices, embedding_table):
    flat_idx = action_indices.reshape(-1).astype(jnp.int32)
    out = _gather_kernel(flat_idx, embedding_table.reshape(-1))
    return jnp.transpose(out, (2, 0, 1))


# speedup vs baseline: 10.0935x; 5.4726x over previous
import functools

import jax
import jax.numpy as jnp
from jax import lax
from jax.experimental import pallas as pl
from jax.experimental.pallas import tpu as pltpu
from jax.experimental.pallas import tpu_sc as plsc

V = 52
D = 64
NB = 16384
S = 50
NC, NS = 2, 16
NW = NC * NS
B_PER_W = NB // NW
GB = 4
L = 16

_mesh = plsc.VectorSubcoreMesh(core_axis_name="c", subcore_axis_name="s")


@functools.partial(
    pl.kernel,
    out_type=jax.ShapeDtypeStruct((S, D, NB), jnp.float32),
    mesh=_mesh,
    compiler_params=pltpu.CompilerParams(needs_layout_passes=False),
    scratch_types=[
        pltpu.VMEM((B_PER_W * S,), jnp.int32),
        pltpu.VMEM((V * (D + 1),), jnp.float32),
        pltpu.VMEM((D, 128), jnp.float32),
        pltpu.VMEM((D, 128), jnp.float32),
        pltpu.SemaphoreType.DMA,
        pltpu.SemaphoreType.DMA,
    ],
)
def _gather_kernel(idx_hbm, table_hbm, out_hbm, idx_v, tab_v,
                   stage0, stage1, ssem0, ssem1):
    wid = lax.axis_index("s") * NC + lax.axis_index("c")
    base = pl.multiple_of(wid * (B_PER_W * S), B_PER_W * S)
    pltpu.sync_copy(idx_hbm.at[pl.ds(base, B_PER_W * S)], idx_v)
    pltpu.sync_copy(table_hbm, tab_v)
    stage = (stage0, stage1)
    ssem = (ssem0, ssem1)

    iota = lax.iota(jnp.int32, L) * S

    def store(s, gb, half):
        b0 = (wid * GB + gb) * 128
        return pltpu.make_async_copy(
            stage[half], out_hbm.at[s, :, pl.ds(pl.multiple_of(b0, 128), 128)],
            ssem[half])

    def gb_body(gb, carry):
        def body(t, carry):
            for half in range(2):
                s = 2 * t + half

                @pl.when(t >= 1)
                def _(s=s, half=half):
                    store(s - 2, gb, half).wait()

                for g in range(8):
                    off0 = gb * (128 * S) + g * (L * S) + s
                    idx16 = plsc.load_gather(idx_v, [iota + off0])
                    rowb = idx16 * (D + 1)

                    @plsc.parallel_loop(0, D, unroll=8)
                    def _(d, g=g, half=half, rowb=rowb):
                        val = plsc.load_gather(tab_v, [rowb + d])
                        stage[half][d, pl.ds(g * L, L)] = val

                store(s, gb, half).start()
            return carry

        lax.fori_loop(0, S // 2, body, 0)
        store(S - 2, gb, 0).wait()
        store(S - 1, gb, 1).wait()
        return carry

    lax.fori_loop(0, GB, gb_body, 0)


def kernel(action_indices, embedding_table):
    flat_idx = action_indices.reshape(-1).astype(jnp.int32)
    table_padded = jnp.pad(embedding_table, ((0, 0), (0, 1))).reshape(-1)
    out = _gather_kernel(flat_idx, table_padded)
    return jnp.transpose(out, (2, 0, 1))


# speedup vs baseline: 11.2246x; 1.1121x over previous
import functools

import jax
import jax.numpy as jnp
from jax import lax
from jax.experimental import pallas as pl
from jax.experimental.pallas import tpu as pltpu
from jax.experimental.pallas import tpu_sc as plsc

V = 52
D = 64
NB = 16384
S = 50
NC, NS = 2, 16
NW = NC * NS
B_PER_W = NB // NW
GB = 4
L = 16

_mesh = plsc.VectorSubcoreMesh(core_axis_name="c", subcore_axis_name="s")


@functools.partial(
    pl.kernel,
    out_type=jax.ShapeDtypeStruct((S, D, NB), jnp.float32),
    mesh=_mesh,
    compiler_params=pltpu.CompilerParams(needs_layout_passes=False),
    scratch_types=[
        pltpu.VMEM((S * B_PER_W,), jnp.int32),
        pltpu.VMEM((V * (D + 1),), jnp.float32),
        pltpu.VMEM((D, 128), jnp.float32),
        pltpu.VMEM((D, 128), jnp.float32),
        pltpu.SemaphoreType.DMA,
        pltpu.SemaphoreType.DMA,
        pltpu.SemaphoreType.DMA,
    ],
)
def _gather_kernel(idx_hbm, table_hbm, out_hbm, idx_v, tab_v,
                   stage0, stage1, isem, ssem0, ssem1):
    wid = lax.axis_index("s") * NC + lax.axis_index("c")
    stage = (stage0, stage1)
    ssem = (ssem0, ssem1)

    def idx_copy(s):
        src = pl.multiple_of(s * NB + wid * B_PER_W, B_PER_W)
        dst = pl.multiple_of(s * B_PER_W, B_PER_W)
        return pltpu.make_async_copy(idx_hbm.at[pl.ds(src, B_PER_W)],
                                     idx_v.at[pl.ds(dst, B_PER_W)], isem)

    def fire(s, carry):
        idx_copy(s).start()
        return carry

    def drain(s, carry):
        idx_copy(s).wait()
        return carry

    lax.fori_loop(0, S, fire, 0)
    pltpu.sync_copy(table_hbm, tab_v)
    lax.fori_loop(0, S, drain, 0)

    def store(s, gb, half):
        b0 = (wid * GB + gb) * 128
        return pltpu.make_async_copy(
            stage[half], out_hbm.at[s, :, pl.ds(pl.multiple_of(b0, 128), 128)],
            ssem[half])

    def gb_body(gb, carry):
        def body(t, carry):
            for half in range(2):
                s = 2 * t + half

                @pl.when(t >= 1)
                def _(s=s, half=half):
                    store(s - 2, gb, half).wait()

                for g in range(8):
                    off = pl.multiple_of(s * B_PER_W + gb * 128 + g * L, L)
                    idx16 = idx_v[pl.ds(off, L)]
                    rowb = idx16 * (D + 1)

                    @plsc.parallel_loop(0, D, unroll=8)
                    def _(d, g=g, half=half, rowb=rowb):
                        val = plsc.load_gather(tab_v, [rowb + d])
                        stage[half][d, pl.ds(g * L, L)] = val

                store(s, gb, half).start()
            return carry

        lax.fori_loop(0, S // 2, body, 0)
        store(S - 2, gb, 0).wait()
        store(S - 1, gb, 1).wait()
        return carry

    lax.fori_loop(0, GB, gb_body, 0)


def kernel(action_indices, embedding_table):
    idx_sm = action_indices.T.reshape(-1).astype(jnp.int32)
    table_padded = jnp.pad(embedding_table, ((0, 0), (0, 1))).reshape(-1)
    out = _gather_kernel(idx_sm, table_padded)
    return jnp.transpose(out, (2, 0, 1))
